# baseline (speedup 1.0000x reference)
import functools

import jax
import jax.numpy as jnp
from jax import lax
from jax.experimental import pallas as pl
from jax.experimental.pallas import tpu as pltpu
from jax.experimental.pallas import tpu_sc as plsc

N_PAD = 10240
IN_CH = 128
OUT_CH = 128
HALF = 64
N_TILES = 16
CHUNK = 128
LANES = 16
ROWS_PER_TILE = N_PAD // N_TILES
ROW_CHUNK = 128
N_ITER = 3
NRING = 2
ESLOTS = 2 * NRING


def _matmul_body(x_ref, w_ref, out_ref):
    y = jnp.dot(x_ref[...], w_ref[...], preferred_element_type=jnp.float32)
    out_ref[0] = y[:, :HALF]
    out_ref[1] = y[:, HALF:]


def _matmul(features, weight):
    m = features.shape[0]
    blk = 1024
    return pl.pallas_call(
        _matmul_body,
        grid=(m // blk,),
        in_specs=[
            pl.BlockSpec((blk, IN_CH), lambda i: (i, 0)),
            pl.BlockSpec((IN_CH, OUT_CH), lambda i: (0, 0)),
        ],
        out_specs=pl.BlockSpec((2, blk, HALF), lambda i: (0, i, 0)),
        out_shape=jax.ShapeDtypeStruct((2, m, HALF), jnp.float32),
    )(features, weight)


def _make_spmm(n_chunks):
    mesh = plsc.VectorSubcoreMesh(core_axis_name="c", subcore_axis_name="s")

    @functools.partial(
        pl.kernel,
        out_type=jax.ShapeDtypeStruct((2, N_PAD, HALF), jnp.float32),
        mesh=mesh,
        compiler_params=pltpu.CompilerParams(
            use_tc_tiling_on_sc=False, needs_layout_passes=False),
        scratch_types=[
            pltpu.VMEM((ESLOTS, 2, CHUNK), jnp.int32),
            pltpu.VMEM((ESLOTS, CHUNK, LANES), jnp.float32),
            pltpu.VMEM((CHUNK, HALF), jnp.float32),
            pltpu.VMEM((CHUNK, HALF), jnp.float32),
            pltpu.VMEM_SHARED((N_PAD, HALF), jnp.float32),
            pltpu.VMEM_SHARED((N_PAD, HALF), jnp.float32),
            pltpu.SemaphoreType.DMA,
            pltpu.SemaphoreType.DMA,
        ],
    )
    def spmm(x_hbm, idx_hbm, val_hbm, bias_hbm, out_hbm,
             ebuf, vbuf, g0, g1, xbuf, ybuf, esem, gsem):
        gbufs = (g0, g1)
        c = lax.axis_index("c")
        s = lax.axis_index("s")
        r0 = s * ROWS_PER_TILE

        def fill_g0(vecs):
            def body(r, carry):
                for q in range(HALF // LANES):
                    g0[r, pl.ds(q * LANES, LANES)] = vecs[q]
                return carry
            lax.fori_loop(0, CHUNK, body, 0)

        for k in range(ROWS_PER_TILE // ROW_CHUNK):
            rr = r0 + k * ROW_CHUNK
            pltpu.sync_copy(x_hbm.at[c, pl.ds(rr, ROW_CHUNK)], g0)
            pltpu.sync_copy(g0, xbuf.at[pl.ds(rr, ROW_CHUNK)])

        def issue_edges(j):
            pltpu.async_copy(idx_hbm.at[s, j], ebuf.at[j % ESLOTS], esem)
            pltpu.async_copy(val_hbm.at[s, j], vbuf.at[j % ESLOTS], esem)

        def wait_edges(j):
            pltpu.make_async_copy(idx_hbm.at[s, j],
                                  ebuf.at[j % ESLOTS], esem).wait()
            pltpu.make_async_copy(val_hbm.at[s, j],
                                  vbuf.at[j % ESLOTS], esem).wait()

        def issue_gather(src, j, b):
            pltpu.async_copy(src.at[ebuf.at[j % ESLOTS, 0]],
                             gbufs[b], gsem)

        def wait_gather(src, j, b):
            pltpu.make_async_copy(src.at[ebuf.at[j % ESLOTS, 0]],
                                  gbufs[b], gsem).wait()

        hops = [(xbuf, ybuf), (ybuf, xbuf), (xbuf, ybuf)]
        for it in range(N_ITER):
            src, dst = hops[it]
            if it == N_ITER - 1:
                pltpu.sync_copy(bias_hbm.at[c],
                                g0.at[pl.ds(0, 1), pl.ds(0, HALF)])
                bvecs = [g0[0, pl.ds(q * LANES, LANES)]
                         for q in range(HALF // LANES)]
                fill_g0(bvecs)
            else:
                fill_g0([jnp.zeros((LANES,), jnp.float32)]
                        * (HALF // LANES))
            for k in range(ROWS_PER_TILE // ROW_CHUNK):
                rr = r0 + k * ROW_CHUNK
                pltpu.sync_copy(g0, dst.at[pl.ds(rr, ROW_CHUNK)])
            plsc.subcore_barrier()

            def process(j, b, guarded):
                wait_gather(src, j, b)

                def scale_group(g, inner):
                    base = g * LANES
                    for e in range(LANES):
                        sv = vbuf[j % ESLOTS, base + e, pl.ds(0, LANES)]
                        row = base + e
                        for q in range(HALF // LANES):
                            sl = gbufs[b][row, pl.ds(q * LANES, LANES)]
                            gbufs[b][row, pl.ds(q * LANES, LANES)] = sl * sv
                    return inner

                lax.fori_loop(0, CHUNK // LANES, scale_group, 0)
                pltpu.sync_copy(gbufs[b], dst.at[ebuf.at[j % ESLOTS, 1]],
                                add=True)

                def refill_gather():
                    wait_edges(j + NRING)
                    issue_gather(src, j + NRING, b)

                if guarded:
                    pl.when(j + NRING < n_chunks)(refill_gather)
                else:
                    refill_gather()

                def refill_edges():
                    issue_edges(j + ESLOTS)

                if guarded:
                    pl.when(j + ESLOTS < n_chunks)(refill_edges)
                else:
                    refill_edges()

            for j in range(ESLOTS):
                issue_edges(j)
            for b in range(NRING):
                wait_edges(b)
                issue_gather(src, b, b)
            for b in range(NRING):
                process(b, b, guarded=False)

            def main_body(g, carry):
                for b in range(NRING):
                    j = NRING * g + b
                    pl.when(j < n_chunks)(
                        functools.partial(process, j, b, True))
                return carry

            lax.fori_loop(1, -(-n_chunks // NRING), main_body, 0)
            plsc.subcore_barrier()

        final = hops[N_ITER - 1][1]
        for k in range(ROWS_PER_TILE // ROW_CHUNK):
            rr = r0 + k * ROW_CHUNK
            pltpu.sync_copy(final.at[pl.ds(rr, ROW_CHUNK)], g0)
            pltpu.sync_copy(g0, out_hbm.at[c, pl.ds(rr, ROW_CHUNK)])

    return spmm


@jax.jit
def kernel(adj_indices, adj_values, features, weight_matrix, bias):
    rows = adj_indices[0].astype(jnp.int32)
    cols = adj_indices[1].astype(jnp.int32)
    vals = adj_values.astype(jnp.float32)
    n_edges = rows.shape[0]
    per_tile = -(-n_edges // (N_TILES * CHUNK)) * CHUNK
    n_chunks = per_tile // CHUNK
    pad = per_tile * N_TILES - n_edges
    rows = jnp.pad(rows, (0, pad)).reshape(N_TILES, n_chunks, CHUNK)
    cols = jnp.pad(cols, (0, pad)).reshape(N_TILES, n_chunks, CHUNK)
    vals = jnp.pad(vals, (0, pad)).reshape(N_TILES, n_chunks, CHUNK)
    idx = jnp.stack([cols, rows], axis=2)
    val16 = jnp.broadcast_to(vals[..., None],
                             (N_TILES, n_chunks, CHUNK, LANES))

    n = features.shape[0]
    feats = jnp.pad(features, ((0, N_PAD - n), (0, 0)))
    x_split = _matmul(feats, weight_matrix)
    bias2 = bias.reshape(2, 1, HALF).astype(jnp.float32)
    out = _make_spmm(n_chunks)(x_split, idx, val16, bias2)
    return out[:, :n].transpose(1, 0, 2).reshape(n, OUT_CH)


# speedup vs baseline: 1.2577x; 1.2577x over previous
import functools

import jax
import jax.numpy as jnp
from jax import lax
from jax.experimental import pallas as pl
from jax.experimental.pallas import tpu as pltpu
from jax.experimental.pallas import tpu_sc as plsc

N_PAD = 10240
IN_CH = 128
OUT_CH = 128
HALF = 64
N_TILES = 16
CHUNK = 128
LANES = 16
ROWS_PER_TILE = N_PAD // N_TILES
ROW_CHUNK = 128
N_ITER = 3
NRING = 2
ESLOTS = 2 * NRING


def _matmul_body(x_ref, w_ref, out_ref):
    y = jnp.dot(x_ref[...], w_ref[...], preferred_element_type=jnp.float32)
    out_ref[0] = y[:, :HALF]
    out_ref[1] = y[:, HALF:]


def _matmul(features, weight):
    m = features.shape[0]
    blk = 1024
    return pl.pallas_call(
        _matmul_body,
        grid=(m // blk,),
        in_specs=[
            pl.BlockSpec((blk, IN_CH), lambda i: (i, 0)),
            pl.BlockSpec((IN_CH, OUT_CH), lambda i: (0, 0)),
        ],
        out_specs=pl.BlockSpec((2, blk, HALF), lambda i: (0, i, 0)),
        out_shape=jax.ShapeDtypeStruct((2, m, HALF), jnp.float32),
    )(features, weight)


def _make_spmm(n_chunks):
    mesh = plsc.VectorSubcoreMesh(core_axis_name="c", subcore_axis_name="s")

    @functools.partial(
        pl.kernel,
        out_type=jax.ShapeDtypeStruct((2, N_PAD, HALF), jnp.float32),
        mesh=mesh,
        compiler_params=pltpu.CompilerParams(
            use_tc_tiling_on_sc=False, needs_layout_passes=False),
        scratch_types=[
            pltpu.VMEM((ESLOTS, 2, CHUNK), jnp.int32),
            pltpu.VMEM((ESLOTS, CHUNK, LANES), jnp.float32),
            pltpu.VMEM((CHUNK, HALF), jnp.float32),
            pltpu.VMEM((CHUNK, HALF), jnp.float32),
            pltpu.VMEM_SHARED((N_PAD, HALF), jnp.float32),
            pltpu.VMEM_SHARED((N_PAD, HALF), jnp.float32),
            pltpu.SemaphoreType.DMA,
            pltpu.SemaphoreType.DMA,
        ],
    )
    def spmm(x_hbm, idx_hbm, val_hbm, bias_hbm, out_hbm,
             ebuf, vbuf, g0, g1, xbuf, ybuf, esem, gsem):
        gbufs = (g0, g1)
        c = lax.axis_index("c")
        s = lax.axis_index("s")
        r0 = s * ROWS_PER_TILE

        def fill_g0(vecs):
            def body(r, carry):
                for q in range(HALF // LANES):
                    g0[r, pl.ds(q * LANES, LANES)] = vecs[q]
                return carry
            lax.fori_loop(0, CHUNK, body, 0)

        for k in range(ROWS_PER_TILE // ROW_CHUNK):
            rr = r0 + k * ROW_CHUNK
            pltpu.sync_copy(x_hbm.at[c, pl.ds(rr, ROW_CHUNK)], g0)
            pltpu.sync_copy(g0, xbuf.at[pl.ds(rr, ROW_CHUNK)])

        def issue_edges(j):
            pltpu.async_copy(idx_hbm.at[s, j], ebuf.at[j % ESLOTS], esem)
            pltpu.async_copy(val_hbm.at[s, j], vbuf.at[j % ESLOTS], esem)

        def wait_edges(j):
            pltpu.make_async_copy(idx_hbm.at[s, j],
                                  ebuf.at[j % ESLOTS], esem).wait()
            pltpu.make_async_copy(val_hbm.at[s, j],
                                  vbuf.at[j % ESLOTS], esem).wait()

        def issue_gather(src, j, b):
            pltpu.async_copy(src.at[ebuf.at[j % ESLOTS, 0]],
                             gbufs[b], gsem)

        def wait_gather(src, j, b):
            pltpu.make_async_copy(src.at[ebuf.at[j % ESLOTS, 0]],
                                  gbufs[b], gsem).wait()

        hops = [(xbuf, ybuf), (ybuf, xbuf), (xbuf, ybuf)]
        for it in range(N_ITER):
            src, dst = hops[it]
            if it == N_ITER - 1:
                pltpu.sync_copy(bias_hbm.at[c],
                                g0.at[pl.ds(0, 1), pl.ds(0, HALF)])
                bvecs = [g0[0, pl.ds(q * LANES, LANES)]
                         for q in range(HALF // LANES)]
                fill_g0(bvecs)
            else:
                fill_g0([jnp.zeros((LANES,), jnp.float32)]
                        * (HALF // LANES))
            for k in range(ROWS_PER_TILE // ROW_CHUNK):
                rr = r0 + k * ROW_CHUNK
                pltpu.sync_copy(g0, dst.at[pl.ds(rr, ROW_CHUNK)])
            plsc.subcore_barrier()

            def process(j, b, guarded):
                wait_gather(src, j, b)

                @plsc.parallel_loop(0, CHUNK, LANES)
                def scale_group(base):
                    for e in range(LANES):
                        sv = vbuf[j % ESLOTS, base + e, pl.ds(0, LANES)]
                        row = base + e
                        for q in range(HALF // LANES):
                            sl = gbufs[b][row, pl.ds(q * LANES, LANES)]
                            gbufs[b][row, pl.ds(q * LANES, LANES)] = sl * sv
                pltpu.sync_copy(gbufs[b], dst.at[ebuf.at[j % ESLOTS, 1]],
                                add=True)

                def refill_gather():
                    wait_edges(j + NRING)
                    issue_gather(src, j + NRING, b)

                if guarded:
                    pl.when(j + NRING < n_chunks)(refill_gather)
                else:
                    refill_gather()

                def refill_edges():
                    issue_edges(j + ESLOTS)

                if guarded:
                    pl.when(j + ESLOTS < n_chunks)(refill_edges)
                else:
                    refill_edges()

            for j in range(ESLOTS):
                issue_edges(j)
            for b in range(NRING):
                wait_edges(b)
                issue_gather(src, b, b)
            for b in range(NRING):
                process(b, b, guarded=False)

            def main_body(g, carry):
                for b in range(NRING):
                    j = NRING * g + b
                    pl.when(j < n_chunks)(
                        functools.partial(process, j, b, True))
                return carry

            lax.fori_loop(1, -(-n_chunks // NRING), main_body, 0)
            plsc.subcore_barrier()

        final = hops[N_ITER - 1][1]
        for k in range(ROWS_PER_TILE // ROW_CHUNK):
            rr = r0 + k * ROW_CHUNK
            pltpu.sync_copy(final.at[pl.ds(rr, ROW_CHUNK)], g0)
            pltpu.sync_copy(g0, out_hbm.at[c, pl.ds(rr, ROW_CHUNK)])

    return spmm


@jax.jit
def kernel(adj_indices, adj_values, features, weight_matrix, bias):
    rows = adj_indices[0].astype(jnp.int32)
    cols = adj_indices[1].astype(jnp.int32)
    vals = adj_values.astype(jnp.float32)
    n_edges = rows.shape[0]
    per_tile = -(-n_edges // (N_TILES * CHUNK)) * CHUNK
    n_chunks = per_tile // CHUNK
    pad = per_tile * N_TILES - n_edges
    rows = jnp.pad(rows, (0, pad)).reshape(N_TILES, n_chunks, CHUNK)
    cols = jnp.pad(cols, (0, pad)).reshape(N_TILES, n_chunks, CHUNK)
    vals = jnp.pad(vals, (0, pad)).reshape(N_TILES, n_chunks, CHUNK)
    idx = jnp.stack([cols, rows], axis=2)
    val16 = jnp.broadcast_to(vals[..., None],
                             (N_TILES, n_chunks, CHUNK, LANES))

    n = features.shape[0]
    feats = jnp.pad(features, ((0, N_PAD - n), (0, 0)))
    x_split = _matmul(feats, weight_matrix)
    bias2 = bias.reshape(2, 1, HALF).astype(jnp.float32)
    out = _make_spmm(n_chunks)(x_split, idx, val16, bias2)
    return out[:, :n].transpose(1, 0, 2).reshape(n, OUT_CH)


# speedup vs baseline: 1.2814x; 1.0188x over previous
import functools

import jax
import jax.numpy as jnp
from jax import lax
from jax.experimental import pallas as pl
from jax.experimental.pallas import tpu as pltpu
from jax.experimental.pallas import tpu_sc as plsc

N_PAD = 10240
IN_CH = 128
OUT_CH = 128
HALF = 64
N_TILES = 16
CHUNK = 128
LANES = 16
ROWS_PER_TILE = N_PAD // N_TILES
ROW_CHUNK = 128
N_ITER = 3
NRING = 2
ESLOTS = 2 * NRING


def _matmul_body(x_ref, w_ref, out_ref):
    y = jnp.dot(x_ref[...], w_ref[...], preferred_element_type=jnp.float32)
    out_ref[0] = y[:, :HALF]
    out_ref[1] = y[:, HALF:]


def _matmul(features, weight):
    m = features.shape[0]
    blk = 1024
    return pl.pallas_call(
        _matmul_body,
        grid=(m // blk,),
        in_specs=[
            pl.BlockSpec((blk, IN_CH), lambda i: (i, 0)),
            pl.BlockSpec((IN_CH, OUT_CH), lambda i: (0, 0)),
        ],
        out_specs=pl.BlockSpec((2, blk, HALF), lambda i: (0, i, 0)),
        out_shape=jax.ShapeDtypeStruct((2, m, HALF), jnp.float32),
    )(features, weight)


def _make_spmm(n_chunks):
    mesh = plsc.VectorSubcoreMesh(core_axis_name="c", subcore_axis_name="s")

    @functools.partial(
        pl.kernel,
        out_type=jax.ShapeDtypeStruct((2, N_PAD, HALF), jnp.float32),
        mesh=mesh,
        compiler_params=pltpu.CompilerParams(
            use_tc_tiling_on_sc=False, needs_layout_passes=False),
        scratch_types=[
            pltpu.VMEM((ESLOTS, 2, CHUNK), jnp.int32),
            pltpu.VMEM((ESLOTS, CHUNK, LANES), jnp.float32),
            pltpu.VMEM((CHUNK, HALF), jnp.float32),
            pltpu.VMEM((CHUNK, HALF), jnp.float32),
            pltpu.VMEM_SHARED((N_PAD, HALF), jnp.float32),
            pltpu.VMEM_SHARED((N_PAD, HALF), jnp.float32),
            pltpu.SemaphoreType.DMA,
            pltpu.SemaphoreType.DMA,
        ],
    )
    def spmm(x_hbm, idx_hbm, val_hbm, bias_hbm, out_hbm,
             ebuf, vbuf, g0, g1, xbuf, ybuf, esem, gsem):
        gbufs = (g0, g1)
        c = lax.axis_index("c")
        s = lax.axis_index("s")
        r0 = s * ROWS_PER_TILE

        def fill_g0(vecs):
            def body(r, carry):
                for q in range(HALF // LANES):
                    g0[r, pl.ds(q * LANES, LANES)] = vecs[q]
                return carry
            lax.fori_loop(0, CHUNK, body, 0)

        for k in range(ROWS_PER_TILE // ROW_CHUNK):
            rr = r0 + k * ROW_CHUNK
            pltpu.sync_copy(x_hbm.at[c, pl.ds(rr, ROW_CHUNK)], g0)
            pltpu.sync_copy(g0, xbuf.at[pl.ds(rr, ROW_CHUNK)])

        def issue_edges(j):
            pltpu.async_copy(idx_hbm.at[s, j], ebuf.at[j % ESLOTS], esem)
            pltpu.async_copy(val_hbm.at[s, j], vbuf.at[j % ESLOTS], esem)

        def wait_edges(j):
            pltpu.make_async_copy(idx_hbm.at[s, j],
                                  ebuf.at[j % ESLOTS], esem).wait()
            pltpu.make_async_copy(val_hbm.at[s, j],
                                  vbuf.at[j % ESLOTS], esem).wait()

        def issue_gather(src, j, b):
            pltpu.async_copy(src.at[ebuf.at[j % ESLOTS, 0]],
                             gbufs[b], gsem)

        def wait_gather(src, j, b):
            pltpu.make_async_copy(src.at[ebuf.at[j % ESLOTS, 0]],
                                  gbufs[b], gsem).wait()

        hops = [(xbuf, ybuf), (ybuf, xbuf), (xbuf, ybuf)]
        for it in range(N_ITER):
            src, dst = hops[it]
            if it == N_ITER - 1:
                pltpu.sync_copy(bias_hbm.at[c],
                                g0.at[pl.ds(0, 1), pl.ds(0, HALF)])
                bvecs = [g0[0, pl.ds(q * LANES, LANES)]
                         for q in range(HALF // LANES)]
                fill_g0(bvecs)
            else:
                fill_g0([jnp.zeros((LANES,), jnp.float32)]
                        * (HALF // LANES))
            for k in range(ROWS_PER_TILE // ROW_CHUNK):
                rr = r0 + k * ROW_CHUNK
                pltpu.sync_copy(g0, dst.at[pl.ds(rr, ROW_CHUNK)])
            plsc.subcore_barrier()

            def process(j, b, guarded):
                wait_gather(src, j, b)

                @plsc.parallel_loop(0, CHUNK, LANES, unroll=2)
                def scale_group(base):
                    for e in range(LANES):
                        sv = vbuf[j % ESLOTS, base + e, pl.ds(0, LANES)]
                        row = base + e
                        for q in range(HALF // LANES):
                            sl = gbufs[b][row, pl.ds(q * LANES, LANES)]
                            gbufs[b][row, pl.ds(q * LANES, LANES)] = sl * sv
                pltpu.sync_copy(gbufs[b], dst.at[ebuf.at[j % ESLOTS, 1]],
                                add=True)

                def refill_gather():
                    wait_edges(j + NRING)
                    issue_gather(src, j + NRING, b)

                if guarded:
                    pl.when(j + NRING < n_chunks)(refill_gather)
                else:
                    refill_gather()

                def refill_edges():
                    issue_edges(j + ESLOTS)

                if guarded:
                    pl.when(j + ESLOTS < n_chunks)(refill_edges)
                else:
                    refill_edges()

            for j in range(ESLOTS):
                issue_edges(j)
            for b in range(NRING):
                wait_edges(b)
                issue_gather(src, b, b)
            for b in range(NRING):
                process(b, b, guarded=False)

            def main_body(g, carry):
                for b in range(NRING):
                    j = NRING * g + b
                    pl.when(j < n_chunks)(
                        functools.partial(process, j, b, True))
                return carry

            lax.fori_loop(1, -(-n_chunks // NRING), main_body, 0)
            plsc.subcore_barrier()

        final = hops[N_ITER - 1][1]
        for k in range(ROWS_PER_TILE // ROW_CHUNK):
            rr = r0 + k * ROW_CHUNK
            pltpu.sync_copy(final.at[pl.ds(rr, ROW_CHUNK)], g0)
            pltpu.sync_copy(g0, out_hbm.at[c, pl.ds(rr, ROW_CHUNK)])

    return spmm


@jax.jit
def kernel(adj_indices, adj_values, features, weight_matrix, bias):
    rows = adj_indices[0].astype(jnp.int32)
    cols = adj_indices[1].astype(jnp.int32)
    vals = adj_values.astype(jnp.float32)
    n_edges = rows.shape[0]
    per_tile = -(-n_edges // (N_TILES * CHUNK)) * CHUNK
    n_chunks = per_tile // CHUNK
    pad = per_tile * N_TILES - n_edges
    rows = jnp.pad(rows, (0, pad)).reshape(N_TILES, n_chunks, CHUNK)
    cols = jnp.pad(cols, (0, pad)).reshape(N_TILES, n_chunks, CHUNK)
    vals = jnp.pad(vals, (0, pad)).reshape(N_TILES, n_chunks, CHUNK)
    idx = jnp.stack([cols, rows], axis=2)
    val16 = jnp.broadcast_to(vals[..., None],
                             (N_TILES, n_chunks, CHUNK, LANES))

    n = features.shape[0]
    feats = jnp.pad(features, ((0, N_PAD - n), (0, 0)))
    x_split = _matmul(feats, weight_matrix)
    bias2 = bias.reshape(2, 1, HALF).astype(jnp.float32)
    out = _make_spmm(n_chunks)(x_split, idx, val16, bias2)
    return out[:, :n].transpose(1, 0, 2).reshape(n, OUT_CH)
